# baseline (speedup 1.0000x reference)
import jax
import jax.numpy as jnp
from jax import lax
from jax.experimental import pallas as pl
from jax.experimental.pallas import tpu as pltpu
from jax.experimental.pallas import tpu_sc as plsc

_B = 8
_N = 5000
_LANES = 16
_NP = 5136
_CHUNKS = _NP // _LANES
_NUM_DET = 3
_IOU_THRESH = 0.5
_BIG = 2**30


def _nms_body(scores_h, x1_h, y1_h, x2_h, y2_h, cls_h, out_h,
              s_v, x1_v, y1_v, x2_v, y2_v, c_v, o_v):
    wid = lax.axis_index("s") * 2 + lax.axis_index("c")

    @pl.when(wid < _B)
    def _():
        pltpu.sync_copy(scores_h.at[wid], s_v)
        pltpu.sync_copy(x1_h.at[wid], x1_v)
        pltpu.sync_copy(y1_h.at[wid], y1_v)
        pltpu.sync_copy(x2_h.at[wid], x2_v)
        pltpu.sync_copy(y2_h.at[wid], y2_v)
        pltpu.sync_copy(cls_h.at[wid], c_v)

        lanes = lax.iota(jnp.int32, _LANES)
        neg = jnp.float32(-jnp.inf)

        def select_pass(prev):
            def chunk(i, carry):
                bv, bi = carry
                sl = pl.ds(i * _LANES, _LANES)
                idx = i * _LANES + lanes
                s = s_v[sl]
                if prev is not None:
                    ib, wx1, wy1, wx2, wy2, wcls, warea = prev
                    bx1 = x1_v[sl]
                    by1 = y1_v[sl]
                    bx2 = x2_v[sl]
                    by2 = y2_v[sl]
                    cc = c_v[sl]
                    ix1 = jnp.maximum(wx1, bx1)
                    iy1 = jnp.maximum(wy1, by1)
                    ix2 = jnp.minimum(wx2, bx2)
                    iy2 = jnp.minimum(wy2, by2)
                    inter = (jnp.maximum(ix2 - ix1, 0.0)
                             * jnp.maximum(iy2 - iy1, 0.0))
                    area_b = (jnp.maximum(bx2 - bx1, 0.0)
                              * jnp.maximum(by2 - by1, 0.0))
                    iou = inter / jnp.maximum(warea + area_b - inter, 1e-9)
                    kill = ((iou > _IOU_THRESH) & (cc == wcls)) | (idx == ib)
                    s = jnp.where(kill, neg, s)
                    s_v[sl] = s
                take = s > bv
                return jnp.where(take, s, bv), jnp.where(take, idx, bi)

            bv0 = jnp.full((_LANES,), neg, jnp.float32)
            bi0 = jnp.zeros((_LANES,), jnp.int32)
            bv, bi = lax.fori_loop(0, _CHUNKS, chunk, (bv0, bi0))
            mv = bv[0]
            mi = bi[0]
            for j in range(1, _LANES):
                v = bv[j]
                i2 = bi[j]
                better = (v > mv) | ((v == mv) & (i2 < mi))
                mv = jnp.where(better, v, mv)
                mi = jnp.where(better, i2, mi)
            return mi

        def fetch(ib):
            sl = pl.ds(ib, _LANES)
            wx1 = jnp.full((_LANES,), x1_v[sl][0])
            wy1 = jnp.full((_LANES,), y1_v[sl][0])
            wx2 = jnp.full((_LANES,), x2_v[sl][0])
            wy2 = jnp.full((_LANES,), y2_v[sl][0])
            wcls = jnp.full((_LANES,), c_v[sl][0])
            warea = (jnp.maximum(wx2 - wx1, 0.0)
                     * jnp.maximum(wy2 - wy1, 0.0))
            return ib, wx1, wy1, wx2, wy2, wcls, warea

        ib0 = select_pass(None)
        ib1 = select_pass(fetch(ib0))
        ib2 = select_pass(fetch(ib1))

        res = jnp.where(lanes == 0, ib0,
              jnp.where(lanes == 1, ib1,
              jnp.where(lanes == 2, ib2, 0)))
        o_v[...] = res
        pltpu.sync_copy(o_v, out_h.at[wid])


def kernel(scores, boxes, classes):
    pad = ((0, 0), (0, _NP - _N))
    s = jnp.pad(scores, pad, constant_values=-jnp.inf)
    x1 = jnp.pad(boxes[..., 0], pad)
    y1 = jnp.pad(boxes[..., 1], pad)
    x2 = jnp.pad(boxes[..., 2], pad)
    y2 = jnp.pad(boxes[..., 3], pad)
    cl = jnp.pad(classes, pad, constant_values=-1)

    call = pl.kernel(
        _nms_body,
        out_type=jax.ShapeDtypeStruct((_B, _LANES), jnp.int32),
        mesh=plsc.VectorSubcoreMesh(core_axis_name="c", subcore_axis_name="s"),
        scratch_types=[
            pltpu.VMEM((_NP,), jnp.float32),
            pltpu.VMEM((_NP,), jnp.float32),
            pltpu.VMEM((_NP,), jnp.float32),
            pltpu.VMEM((_NP,), jnp.float32),
            pltpu.VMEM((_NP,), jnp.float32),
            pltpu.VMEM((_NP,), jnp.int32),
            pltpu.VMEM((_LANES,), jnp.int32),
        ],
    )
    out = call(s, x1, y1, x2, y2, cl)
    return out[:, :_NUM_DET]


# speedup vs baseline: 1.5395x; 1.5395x over previous
import jax
import jax.numpy as jnp
from jax import lax
from jax.experimental import pallas as pl
from jax.experimental.pallas import tpu as pltpu
from jax.experimental.pallas import tpu_sc as plsc

_B = 8
_N = 5000
_LANES = 16
_NP = 5056
_CHUNKS = _NP // _LANES
_NUM_DET = 3
_ROWS = 6
_UNROLL = 4


def _nms_body(arr_h, out_h, s_v, x1_v, y1_v, x2_v, y2_v, c_v, o_v, sem):
    wid = lax.axis_index("s") * 2 + lax.axis_index("c")

    @pl.when(wid < _B)
    def _():
        bufs = (s_v, x1_v, y1_v, x2_v, y2_v, c_v)
        copies = [pltpu.async_copy(arr_h.at[wid, j], b, sem)
                  for j, b in enumerate(bufs)]
        for cp in copies:
            cp.wait()

        lanes = lax.iota(jnp.int32, _LANES)
        neg = jnp.float32(-jnp.inf)

        def select_pass(prev, store):
            def chunk(i, carry):
                bv, bi = carry
                sl = pl.ds(i * _LANES, _LANES)
                idx = i * _LANES + lanes
                s = s_v[sl]
                if prev is not None:
                    wx1, wy1, wx2, wy2, wcls, warea = prev
                    bx1 = x1_v[sl]
                    by1 = y1_v[sl]
                    bx2 = x2_v[sl]
                    by2 = y2_v[sl]
                    cc = c_v[sl]
                    ix1 = jnp.maximum(wx1, bx1)
                    iy1 = jnp.maximum(wy1, by1)
                    ix2 = jnp.minimum(wx2, bx2)
                    iy2 = jnp.minimum(wy2, by2)
                    inter = (jnp.maximum(ix2 - ix1, 0.0)
                             * jnp.maximum(iy2 - iy1, 0.0))
                    area_b = (jnp.maximum(bx2 - bx1, 0.0)
                              * jnp.maximum(by2 - by1, 0.0))
                    denom = jnp.maximum(warea + area_b - inter, 1e-9)
                    kill = (inter > 0.5 * denom) & (cc == wcls)
                    s = jnp.where(kill, neg, s)
                    if store:
                        s_v[sl] = s
                take = s > bv
                return jnp.where(take, s, bv), jnp.where(take, idx, bi)

            bv0 = jnp.full((_LANES,), neg, jnp.float32)
            bi0 = jnp.zeros((_LANES,), jnp.int32)
            bv, bi = lax.fori_loop(0, _CHUNKS, chunk, (bv0, bi0),
                                   unroll=_UNROLL)
            mv = bv[0]
            mi = bi[0]
            for j in range(1, _LANES):
                v = bv[j]
                i2 = bi[j]
                better = (v > mv) | ((v == mv) & (i2 < mi))
                mv = jnp.where(better, v, mv)
                mi = jnp.where(better, i2, mi)
            return mi

        def fetch(ib):
            sl = pl.ds(ib, _LANES)
            sv = s_v[sl]
            s_v[sl] = jnp.where(lanes == 0, neg, sv)
            wx1 = jnp.full((_LANES,), x1_v[sl][0])
            wy1 = jnp.full((_LANES,), y1_v[sl][0])
            wx2 = jnp.full((_LANES,), x2_v[sl][0])
            wy2 = jnp.full((_LANES,), y2_v[sl][0])
            wcls = jnp.full((_LANES,), c_v[sl][0])
            warea = (jnp.maximum(wx2 - wx1, 0.0)
                     * jnp.maximum(wy2 - wy1, 0.0))
            return wx1, wy1, wx2, wy2, wcls, warea

        ib0 = select_pass(None, False)
        ib1 = select_pass(fetch(ib0), True)
        ib2 = select_pass(fetch(ib1), False)

        res = jnp.where(lanes == 0, ib0,
              jnp.where(lanes == 1, ib1,
              jnp.where(lanes == 2, ib2, 0)))
        o_v[...] = res
        pltpu.sync_copy(o_v, out_h.at[wid])


def kernel(scores, boxes, classes):
    arr = jnp.stack(
        [scores,
         boxes[..., 0], boxes[..., 1], boxes[..., 2], boxes[..., 3],
         classes.astype(jnp.float32)],
        axis=1)
    pad_val = jnp.full((_B, _ROWS, _NP - _N), -jnp.inf, jnp.float32)
    arr = jnp.concatenate([arr, pad_val], axis=2)

    call = pl.kernel(
        _nms_body,
        out_type=jax.ShapeDtypeStruct((_B, _LANES), jnp.int32),
        mesh=plsc.VectorSubcoreMesh(core_axis_name="c", subcore_axis_name="s"),
        scratch_types=[
            pltpu.VMEM((_NP,), jnp.float32),
            pltpu.VMEM((_NP,), jnp.float32),
            pltpu.VMEM((_NP,), jnp.float32),
            pltpu.VMEM((_NP,), jnp.float32),
            pltpu.VMEM((_NP,), jnp.float32),
            pltpu.VMEM((_NP,), jnp.float32),
            pltpu.VMEM((_LANES,), jnp.int32),
            pltpu.SemaphoreType.DMA,
        ],
    )
    out = call(arr)
    return out[:, :_NUM_DET]


# speedup vs baseline: 1.7205x; 1.1176x over previous
import jax
import jax.numpy as jnp
from jax import lax
from jax.experimental import pallas as pl
from jax.experimental.pallas import tpu as pltpu
from jax.experimental.pallas import tpu_sc as plsc

_B = 8
_N = 5000
_LANES = 16
_HALF = 2560
_STRIDE = 2440
_SBUF = 2688
_CHUNKS = 160
_NUM_DET = 3
_UNROLL = 4


def _splat(x):
    return jnp.full((_LANES,), x)


def _nms_body(arr_h, out_h,
              s_v, x1_v, y1_v, x2_v, y2_v, c_v, e_v, p_v, ob_v, o2_v, slab,
              sem_s, sem_bc):
    s = lax.axis_index("s")
    half = jnp.where(s >= _B, 1, 0)
    b = s - _B * half
    start = half * _STRIDE
    partner = jnp.where(s >= _B, s - _B, s + _B)

    cp_s = pltpu.async_copy(arr_h.at[s, 0], s_v.at[pl.ds(0, _HALF)], sem_s)
    rest = [pltpu.async_copy(arr_h.at[s, j], dst.at[pl.ds(0, _HALF)], sem_bc)
            for j, dst in ((1, x1_v), (2, y1_v), (3, x2_v), (4, y2_v),
                           (5, c_v))]

    lanes = lax.iota(jnp.int32, _LANES)
    neg = jnp.float32(-jnp.inf)

    cp_s.wait()

    def select_pass(prev, store):
        def chunk(i, carry):
            bv, bi = carry
            base = i * _LANES
            sl = pl.ds(base, _LANES)
            idx = base + lanes
            sc = s_v[sl]
            if prev is not None:
                wx1, wy1, wx2, wy2, wcls, warea = prev
                bx1 = x1_v[sl]
                by1 = y1_v[sl]
                bx2 = x2_v[sl]
                by2 = y2_v[sl]
                cc = c_v[sl]
                ix1 = jnp.maximum(wx1, bx1)
                iy1 = jnp.maximum(wy1, by1)
                ix2 = jnp.minimum(wx2, bx2)
                iy2 = jnp.minimum(wy2, by2)
                inter = (jnp.maximum(ix2 - ix1, 0.0)
                         * jnp.maximum(iy2 - iy1, 0.0))
                area_b = (bx2 - bx1) * (by2 - by1)
                kill = ((inter > 0.5 * (warea + area_b - inter))
                        & (cc == wcls))
                sc = jnp.where(kill, neg, sc)
                if store:
                    s_v[sl] = sc
            take = sc > bv
            return jnp.where(take, sc, bv), jnp.where(take, idx, bi)

        bv0 = jnp.full((_LANES,), neg, jnp.float32)
        bi0 = jnp.zeros((_LANES,), jnp.int32)
        bv, bi = lax.fori_loop(0, _CHUNKS, chunk, (bv0, bi0),
                               unroll=_UNROLL)
        mv = bv[0]
        mi = bi[0]
        for j in range(1, _LANES):
            v = bv[j]
            i2 = bi[j]
            better = (v > mv) | ((v == mv) & (i2 < mi))
            mv = jnp.where(better, v, mv)
            mi = jnp.where(better, i2, mi)
        return mv, mi

    def exchange(r, mv, mi):
        gi_f = (start + mi).astype(jnp.float32)
        sl = pl.ds(mi, _LANES)
        wx1 = _splat(x1_v[sl][0])
        wy1 = _splat(y1_v[sl][0])
        wx2 = _splat(x2_v[sl][0])
        wy2 = _splat(y2_v[sl][0])
        wcl = _splat(c_v[sl][0])
        wa = (wx2 - wx1) * (wy2 - wy1)
        t = _splat(mv)
        t = jnp.where(lanes == 1, _splat(gi_f), t)
        t = jnp.where(lanes == 2, wx1, t)
        t = jnp.where(lanes == 3, wy1, t)
        t = jnp.where(lanes == 4, wx2, t)
        t = jnp.where(lanes == 5, wy2, t)
        t = jnp.where(lanes == 6, wcl, t)
        t = jnp.where(lanes == 7, wa, t)
        e_v[...] = t
        pltpu.sync_copy(e_v, slab.at[pl.ds((r * 16 + s) * _LANES, _LANES)])
        plsc.subcore_barrier()
        pltpu.sync_copy(slab.at[pl.ds((r * 16 + partner) * _LANES, _LANES)],
                        p_v)
        pv = p_v[...]
        pbetter = (pv[0] > mv) | ((pv[0] == mv) & (pv[1] < gi_f))
        return jnp.where(pbetter, pv, t)

    def winner_splats(wt):
        return (_splat(wt[2]), _splat(wt[3]), _splat(wt[4]), _splat(wt[5]),
                _splat(wt[6]), _splat(wt[7]))

    mv0, mi0 = select_pass(None, False)
    wt0 = exchange(0, mv0, mi0)
    g0 = wt0[1].astype(jnp.int32)

    for cp in rest:
        cp.wait()

    mv1, mi1 = select_pass(winner_splats(wt0), True)
    wt1 = exchange(1, mv1, mi1)
    g1 = wt1[1].astype(jnp.int32)

    mv2, mi2 = select_pass(winner_splats(wt1), False)
    wt2 = exchange(2, mv2, mi2)
    g2 = wt2[1].astype(jnp.int32)

    ov = jnp.where(lanes == 0, g0,
         jnp.where(lanes == 1, g1,
         jnp.where(lanes == 2, g2, 0)))
    e_v[...] = ov.astype(jnp.float32)
    pltpu.sync_copy(e_v, slab.at[pl.ds((3 * 16 + s) * _LANES, _LANES)])
    plsc.subcore_barrier()

    @pl.when(s == 0)
    def _():
        pltpu.sync_copy(slab.at[pl.ds(3 * 16 * _LANES, _B * _LANES)], ob_v)
        r = [ob_v[pl.ds(i * _LANES, _LANES)] for i in range(_B)]
        vals = {}
        for img in range(_B):
            for k in range(_NUM_DET):
                vals[img * _NUM_DET + k] = r[img][k]
        f0 = _splat(vals[0])
        for j in range(1, 16):
            f0 = jnp.where(lanes == j, _splat(vals[j]), f0)
        f1 = _splat(vals[8])
        for j in range(1, 16):
            f1 = jnp.where(lanes == j, _splat(vals[8 + j]), f1)
        o2_v[pl.ds(0, _LANES)] = f0.astype(jnp.int32)
        o2_v[pl.ds(8, _LANES)] = f1.astype(jnp.int32)
        pltpu.sync_copy(o2_v, out_h)


def kernel(scores, boxes, classes):
    planes = jnp.stack(
        [scores,
         boxes[..., 0], boxes[..., 1], boxes[..., 2], boxes[..., 3],
         classes.astype(jnp.float32)],
        axis=1)
    arr = jnp.concatenate(
        [planes[:, :, :_HALF], planes[:, :, _STRIDE:]], axis=0)

    call = pl.kernel(
        _nms_body,
        out_type=jax.ShapeDtypeStruct((_B * _NUM_DET,), jnp.int32),
        mesh=plsc.VectorSubcoreMesh(core_axis_name="c", subcore_axis_name="s",
                                    num_cores=1),
        scratch_types=[
            pltpu.VMEM((_SBUF,), jnp.float32),
            pltpu.VMEM((_SBUF,), jnp.float32),
            pltpu.VMEM((_SBUF,), jnp.float32),
            pltpu.VMEM((_SBUF,), jnp.float32),
            pltpu.VMEM((_SBUF,), jnp.float32),
            pltpu.VMEM((_SBUF,), jnp.float32),
            pltpu.VMEM((_LANES,), jnp.float32),
            pltpu.VMEM((_LANES,), jnp.float32),
            pltpu.VMEM((_B * _LANES,), jnp.float32),
            pltpu.VMEM((_B * _NUM_DET,), jnp.int32),
            pltpu.VMEM_SHARED((4 * 16 * _LANES,), jnp.float32),
            pltpu.SemaphoreType.DMA,
            pltpu.SemaphoreType.DMA,
        ],
    )
    out = call(arr)
    return out.reshape(_B, _NUM_DET)
